# baseline (speedup 1.0000x reference)
import functools

import jax
import jax.numpy as jnp
from jax import lax
from jax.experimental import pallas as pl
from jax.experimental.pallas import tpu as pltpu
from jax.experimental.pallas import tpu_sc as plsc

_NC = 2
_NS = 16
_NW = _NC * _NS

_T = 128


def _scatter_rows(x, pos, out_rows, chunk):
    m, d0, d1 = x.shape
    rows_per_w = m // _NW
    n_chunks = rows_per_w // chunk
    posr = pos.reshape(_NW, n_chunks, chunk)
    mesh = plsc.VectorSubcoreMesh(core_axis_name="c", subcore_axis_name="s")

    @functools.partial(
        pl.kernel,
        mesh=mesh,
        out_type=jax.ShapeDtypeStruct((out_rows, d0, d1), jnp.float32),
        scratch_types=[
            pltpu.VMEM((n_chunks, chunk), jnp.int32),
            pltpu.VMEM((chunk, d0, d1), jnp.float32),
            pltpu.VMEM((chunk, d0, d1), jnp.float32),
            pltpu.VMEM((chunk, d0, d1), jnp.float32),
            pltpu.SemaphoreType.DMA,
            pltpu.SemaphoreType.DMA,
            pltpu.SemaphoreType.DMA,
            pltpu.SemaphoreType.DMA,
            pltpu.SemaphoreType.DMA,
            pltpu.SemaphoreType.DMA,
        ],
    )
    def sk(x_hbm, pos_hbm, out_hbm, idx_v, buf0, buf1, buf2,
           r0, r1, r2, w0, w1, w2):
        wid = lax.axis_index("s") * _NC + lax.axis_index("c")
        base = wid * rows_per_w
        pltpu.sync_copy(pos_hbm.at[wid], idx_v)
        bufs, rsem, wsem = (buf0, buf1, buf2), (r0, r1, r2), (w0, w1, w2)
        reads = [None] * n_chunks
        writes = [None] * n_chunks
        reads[0] = pltpu.async_copy(
            x_hbm.at[pl.ds(base, chunk)], bufs[0], rsem[0]
        )
        if n_chunks > 1:
            reads[1] = pltpu.async_copy(
                x_hbm.at[pl.ds(base + chunk, chunk)], bufs[1], rsem[1]
            )
        for c in range(n_chunks):
            b = c % 3
            reads[c].wait()
            writes[c] = pltpu.async_copy(
                bufs[b], out_hbm.at[idx_v.at[c]], wsem[b]
            )
            if c + 2 < n_chunks:
                nb = (c + 2) % 3
                if c >= 1:
                    writes[c - 1].wait()
                reads[c + 2] = pltpu.async_copy(
                    x_hbm.at[pl.ds(base + (c + 2) * chunk, chunk)],
                    bufs[nb],
                    rsem[nb],
                )
        for c in range(max(0, n_chunks - 3), n_chunks):
            writes[c].wait()

    return sk(x, posr)


def _gather_rows(table, idx, out_rows, chunk):
    d = table.shape[1]
    rows_per_w = out_rows // _NW
    n_chunks = rows_per_w // chunk
    mesh = plsc.VectorSubcoreMesh(core_axis_name="c", subcore_axis_name="s")

    @functools.partial(
        pl.kernel,
        mesh=mesh,
        out_type=jax.ShapeDtypeStruct((out_rows, d), jnp.float32),
        scratch_types=[
            pltpu.VMEM((rows_per_w,), jnp.int32),
            pltpu.VMEM((chunk, d), jnp.float32),
            pltpu.VMEM((chunk, d), jnp.float32),
            pltpu.VMEM((chunk, d), jnp.float32),
            pltpu.SemaphoreType.DMA,
            pltpu.SemaphoreType.DMA,
            pltpu.SemaphoreType.DMA,
            pltpu.SemaphoreType.DMA,
            pltpu.SemaphoreType.DMA,
            pltpu.SemaphoreType.DMA,
        ],
    )
    def gk(table_hbm, idx_hbm, out_hbm, idx_v, buf0, buf1, buf2,
           g0, g1, g2, o0, o1, o2):
        wid = lax.axis_index("s") * _NC + lax.axis_index("c")
        base = wid * rows_per_w
        pltpu.sync_copy(idx_hbm.at[pl.ds(base, rows_per_w)], idx_v)
        bufs, gsem, osem = (buf0, buf1, buf2), (g0, g1, g2), (o0, o1, o2)
        gets = [None] * n_chunks
        puts = [None] * n_chunks
        gets[0] = pltpu.async_copy(
            table_hbm.at[idx_v.at[pl.ds(0, chunk)]], bufs[0], gsem[0]
        )
        if n_chunks > 1:
            gets[1] = pltpu.async_copy(
                table_hbm.at[idx_v.at[pl.ds(chunk, chunk)]], bufs[1], gsem[1]
            )
        for c in range(n_chunks):
            b = c % 3
            gets[c].wait()
            puts[c] = pltpu.async_copy(
                bufs[b], out_hbm.at[pl.ds(base + c * chunk, chunk)], osem[b]
            )
            if c + 2 < n_chunks:
                nb = (c + 2) % 3
                if c >= 1:
                    puts[c - 1].wait()
                gets[c + 2] = pltpu.async_copy(
                    table_hbm.at[idx_v.at[pl.ds((c + 2) * chunk, chunk)]],
                    bufs[nb],
                    gsem[nb],
                )
        for c in range(max(0, n_chunks - 3), n_chunks):
            puts[c].wait()

    return gk(table, idx)


def _mm_body(gid_ref, act_ref, x_ref, w_ref, b_ref, o_ref):
    i = pl.program_id(0)
    max_deg = x_ref.shape[1]

    @pl.when(act_ref[i] == 1)
    def _():
        acc = b_ref[0] + jnp.dot(
            x_ref[:, 0, :], w_ref[0, 0], preferred_element_type=jnp.float32
        )
        for j in range(1, max_deg):
            acc = acc + jnp.dot(
                x_ref[:, j, :], w_ref[0, j], preferred_element_type=jnp.float32
            )
        o_ref[...] = acc


def kernel(neighbour_h, node_types, W, b):
    n, max_deg, h = neighbour_h.shape
    n_type, k_dim, n_dim = W.shape

    nblk = n // _T + n_type
    cap = nblk * _T

    t32 = node_types.astype(jnp.int32)
    onehot = (t32[:, None] == jnp.arange(n_type, dtype=jnp.int32)[None, :]).astype(
        jnp.int32
    )
    counts = onehot.sum(axis=0)
    padded = ((counts + _T - 1) // _T) * _T
    pstart = jnp.cumsum(padded) - padded
    rank = (jnp.cumsum(onehot, axis=0) - onehot)
    rank_t = (rank * onehot).sum(axis=1)
    pos = pstart[t32] + rank_t

    nb_g = padded // _T
    blk_end = jnp.cumsum(nb_g)
    blk = jnp.arange(nblk, dtype=jnp.int32)
    blk_gid = (blk[:, None] >= blk_end[None, :]).astype(jnp.int32).sum(axis=1)
    blk_gid = jnp.minimum(blk_gid, n_type - 1)
    blk_act = (blk < blk_end[n_type - 1]).astype(jnp.int32)

    xs = _scatter_rows(neighbour_h, pos, cap, chunk=16)

    grid_spec = pltpu.PrefetchScalarGridSpec(
        num_scalar_prefetch=2,
        grid=(nblk,),
        in_specs=[
            pl.BlockSpec((_T, max_deg, h), lambda i, g, a: (a[i] * i, 0, 0)),
            pl.BlockSpec((1, max_deg, h, n_dim), lambda i, g, a: (g[i], 0, 0, 0)),
            pl.BlockSpec((1, 1, n_dim), lambda i, g, a: (g[i], 0, 0)),
        ],
        out_specs=pl.BlockSpec((_T, n_dim), lambda i, g, a: (i, 0)),
    )
    ys = pl.pallas_call(
        _mm_body,
        grid_spec=grid_spec,
        out_shape=jax.ShapeDtypeStruct((cap, n_dim), jnp.float32),
        compiler_params=pltpu.CompilerParams(
            dimension_semantics=("arbitrary",),
        ),
    )(blk_gid, blk_act, xs, W.reshape(n_type, max_deg, h, n_dim),
      b.reshape(n_type, 1, n_dim))

    out = _gather_rows(ys, pos, n, chunk=32)
    return out


# speedup vs baseline: 1.1083x; 1.1083x over previous
import functools

import jax
import jax.numpy as jnp
from jax import lax
from jax.experimental import pallas as pl
from jax.experimental.pallas import tpu as pltpu
from jax.experimental.pallas import tpu_sc as plsc

_NC = 2
_NS = 16
_NW = _NC * _NS

_T = 256


def _scatter_rows(x, pos, out_rows, chunk):
    m, d0, d1 = x.shape
    rows_per_w = m // _NW
    n_chunks = rows_per_w // chunk
    posr = pos.reshape(_NW, n_chunks, chunk)
    mesh = plsc.VectorSubcoreMesh(core_axis_name="c", subcore_axis_name="s")

    @functools.partial(
        pl.kernel,
        mesh=mesh,
        out_type=jax.ShapeDtypeStruct((out_rows, d0, d1), jnp.float32),
        scratch_types=[
            pltpu.VMEM((n_chunks, chunk), jnp.int32),
            pltpu.VMEM((chunk, d0, d1), jnp.float32),
            pltpu.VMEM((chunk, d0, d1), jnp.float32),
            pltpu.VMEM((chunk, d0, d1), jnp.float32),
            pltpu.SemaphoreType.DMA,
            pltpu.SemaphoreType.DMA,
            pltpu.SemaphoreType.DMA,
            pltpu.SemaphoreType.DMA,
            pltpu.SemaphoreType.DMA,
            pltpu.SemaphoreType.DMA,
        ],
    )
    def sk(x_hbm, pos_hbm, out_hbm, idx_v, buf0, buf1, buf2,
           r0, r1, r2, w0, w1, w2):
        wid = lax.axis_index("s") * _NC + lax.axis_index("c")
        base = wid * rows_per_w
        pltpu.sync_copy(pos_hbm.at[wid], idx_v)
        bufs, rsem, wsem = (buf0, buf1, buf2), (r0, r1, r2), (w0, w1, w2)
        reads = [None] * n_chunks
        writes = [None] * n_chunks
        reads[0] = pltpu.async_copy(
            x_hbm.at[pl.ds(base, chunk)], bufs[0], rsem[0]
        )
        if n_chunks > 1:
            reads[1] = pltpu.async_copy(
                x_hbm.at[pl.ds(base + chunk, chunk)], bufs[1], rsem[1]
            )
        for c in range(n_chunks):
            b = c % 3
            reads[c].wait()
            writes[c] = pltpu.async_copy(
                bufs[b], out_hbm.at[idx_v.at[c]], wsem[b]
            )
            if c + 2 < n_chunks:
                nb = (c + 2) % 3
                if c >= 1:
                    writes[c - 1].wait()
                reads[c + 2] = pltpu.async_copy(
                    x_hbm.at[pl.ds(base + (c + 2) * chunk, chunk)],
                    bufs[nb],
                    rsem[nb],
                )
        for c in range(max(0, n_chunks - 3), n_chunks):
            writes[c].wait()

    return sk(x, posr)


def _gather_rows(table, idx, out_rows, chunk):
    d = table.shape[1]
    rows_per_w = out_rows // _NW
    n_chunks = rows_per_w // chunk
    mesh = plsc.VectorSubcoreMesh(core_axis_name="c", subcore_axis_name="s")

    @functools.partial(
        pl.kernel,
        mesh=mesh,
        out_type=jax.ShapeDtypeStruct((out_rows, d), jnp.float32),
        scratch_types=[
            pltpu.VMEM((rows_per_w,), jnp.int32),
            pltpu.VMEM((chunk, d), jnp.float32),
            pltpu.VMEM((chunk, d), jnp.float32),
            pltpu.VMEM((chunk, d), jnp.float32),
            pltpu.SemaphoreType.DMA,
            pltpu.SemaphoreType.DMA,
            pltpu.SemaphoreType.DMA,
            pltpu.SemaphoreType.DMA,
            pltpu.SemaphoreType.DMA,
            pltpu.SemaphoreType.DMA,
        ],
    )
    def gk(table_hbm, idx_hbm, out_hbm, idx_v, buf0, buf1, buf2,
           g0, g1, g2, o0, o1, o2):
        wid = lax.axis_index("s") * _NC + lax.axis_index("c")
        base = wid * rows_per_w
        pltpu.sync_copy(idx_hbm.at[pl.ds(base, rows_per_w)], idx_v)
        bufs, gsem, osem = (buf0, buf1, buf2), (g0, g1, g2), (o0, o1, o2)
        gets = [None] * n_chunks
        puts = [None] * n_chunks
        gets[0] = pltpu.async_copy(
            table_hbm.at[idx_v.at[pl.ds(0, chunk)]], bufs[0], gsem[0]
        )
        if n_chunks > 1:
            gets[1] = pltpu.async_copy(
                table_hbm.at[idx_v.at[pl.ds(chunk, chunk)]], bufs[1], gsem[1]
            )
        for c in range(n_chunks):
            b = c % 3
            gets[c].wait()
            puts[c] = pltpu.async_copy(
                bufs[b], out_hbm.at[pl.ds(base + c * chunk, chunk)], osem[b]
            )
            if c + 2 < n_chunks:
                nb = (c + 2) % 3
                if c >= 1:
                    puts[c - 1].wait()
                gets[c + 2] = pltpu.async_copy(
                    table_hbm.at[idx_v.at[pl.ds((c + 2) * chunk, chunk)]],
                    bufs[nb],
                    gsem[nb],
                )
        for c in range(max(0, n_chunks - 3), n_chunks):
            puts[c].wait()

    return gk(table, idx)


def _mm_body(gid_ref, act_ref, x_ref, w_ref, b_ref, o_ref):
    i = pl.program_id(0)
    max_deg = x_ref.shape[1]

    @pl.when(act_ref[i] == 1)
    def _():
        acc = b_ref[0] + jnp.dot(
            x_ref[:, 0, :], w_ref[0, 0], preferred_element_type=jnp.float32
        )
        for j in range(1, max_deg):
            acc = acc + jnp.dot(
                x_ref[:, j, :], w_ref[0, j], preferred_element_type=jnp.float32
            )
        o_ref[...] = acc


def kernel(neighbour_h, node_types, W, b):
    n, max_deg, h = neighbour_h.shape
    n_type, k_dim, n_dim = W.shape

    nblk = n // _T + n_type
    cap = nblk * _T

    t32 = node_types.astype(jnp.int32)
    onehot = (t32[:, None] == jnp.arange(n_type, dtype=jnp.int32)[None, :]).astype(
        jnp.int32
    )
    counts = onehot.sum(axis=0)
    padded = ((counts + _T - 1) // _T) * _T
    pstart = jnp.cumsum(padded) - padded
    rank = (jnp.cumsum(onehot, axis=0) - onehot)
    rank_t = (rank * onehot).sum(axis=1)
    pos = pstart[t32] + rank_t

    nb_g = padded // _T
    blk_end = jnp.cumsum(nb_g)
    blk = jnp.arange(nblk, dtype=jnp.int32)
    blk_gid = (blk[:, None] >= blk_end[None, :]).astype(jnp.int32).sum(axis=1)
    blk_gid = jnp.minimum(blk_gid, n_type - 1)
    blk_act = (blk < blk_end[n_type - 1]).astype(jnp.int32)

    xs = _scatter_rows(neighbour_h, pos, cap, chunk=16)

    grid_spec = pltpu.PrefetchScalarGridSpec(
        num_scalar_prefetch=2,
        grid=(nblk,),
        in_specs=[
            pl.BlockSpec((_T, max_deg, h), lambda i, g, a: (a[i] * i, 0, 0)),
            pl.BlockSpec((1, max_deg, h, n_dim), lambda i, g, a: (g[i], 0, 0, 0)),
            pl.BlockSpec((1, 1, n_dim), lambda i, g, a: (g[i], 0, 0)),
        ],
        out_specs=pl.BlockSpec((_T, n_dim), lambda i, g, a: (i, 0)),
    )
    ys = pl.pallas_call(
        _mm_body,
        grid_spec=grid_spec,
        out_shape=jax.ShapeDtypeStruct((cap, n_dim), jnp.float32),
        compiler_params=pltpu.CompilerParams(
            dimension_semantics=("arbitrary",),
        ),
    )(blk_gid, blk_act, xs, W.reshape(n_type, max_deg, h, n_dim),
      b.reshape(n_type, 1, n_dim))

    out = _gather_rows(ys, pos, n, chunk=32)
    return out
